# baseline (speedup 1.0000x reference)
import functools

import jax
import jax.numpy as jnp
from jax import lax
from jax.experimental import pallas as pl
from jax.experimental.pallas import tpu as pltpu
from jax.experimental.pallas import tpu_sc as plsc

_B = 16384
_D = 64
_NC, _NS = 2, 16
_NW = _NC * _NS
_BPW = _B // _NW
_ICH = 128
_UCH = 256

_MESH = plsc.VectorSubcoreMesh(core_axis_name="c", subcore_axis_name="s")


def _wid():
    return lax.axis_index("s") * _NC + lax.axis_index("c")


def _item_body(iids, it, iout, idx, rows, sem):
    base = _wid() * _BPW
    pltpu.sync_copy(iids.at[pl.ds(base, _BPW)], idx)
    cps = [
        pltpu.async_copy(
            it.at[idx.at[pl.ds(j * _ICH, _ICH)]],
            rows.at[pl.ds(j * _ICH, _ICH)],
            sem,
        )
        for j in range(_BPW // _ICH)
    ]
    for c in cps:
        c.wait()
    pltpu.sync_copy(rows, iout.at[pl.ds(base, _BPW)])


_item_gather = functools.partial(
    pl.kernel,
    mesh=_MESH,
    compiler_params=pltpu.CompilerParams(use_tc_tiling_on_sc=False),
    out_type=jax.ShapeDtypeStruct((_B, _D), jnp.float32),
    scratch_types=[
        pltpu.VMEM((_BPW,), jnp.int32),
        pltpu.VMEM((_BPW, _D), jnp.float32),
        pltpu.SemaphoreType.DMA,
    ],
)(_item_body)


def _user_body(uids, ut, uout, idx, rows, sem):
    base = _wid() * _BPW
    pltpu.sync_copy(uids.at[pl.ds(base, _BPW)], idx)

    def chunk(c, carry):
        cbase = c * _UCH

        def step(g, carry2):
            vec = idx[pl.ds(cbase + g * 16, 16)]
            row = g * 16
            for j in range(16):
                pltpu.async_copy(ut.at[vec[j]], rows.at[row + j], sem)
            return carry2

        lax.fori_loop(0, _UCH // 16, step, 0)
        pltpu.make_async_copy(ut.at[pl.ds(0, _UCH)], rows, sem).wait()
        pltpu.sync_copy(rows, uout.at[pl.ds(base + cbase, _UCH)])
        return carry

    lax.fori_loop(0, _BPW // _UCH, chunk, 0)


_user_gather = functools.partial(
    pl.kernel,
    mesh=_MESH,
    out_type=jax.ShapeDtypeStruct((_B, _D), jnp.float32),
    scratch_types=[
        pltpu.VMEM((_BPW,), jnp.int32),
        pltpu.VMEM((_UCH, _D), jnp.float32),
        pltpu.SemaphoreType.DMA,
    ],
)(_user_body)


def kernel(user_ids, item_ids, user_table, item_table):
    user_emb = _user_gather(user_ids.astype(jnp.int32), user_table)
    item_emb = _item_gather(item_ids.astype(jnp.int32), item_table)
    return (user_emb, item_emb)
